# baseline (speedup 1.0000x reference)
import functools

import jax
import jax.numpy as jnp
from jax import lax
from jax.experimental import pallas as pl
from jax.experimental.pallas import tpu as pltpu
from jax.experimental.pallas import tpu_sc as plsc

DM = 768
NE = 8
NT = 2048
BT = 256
NB = NT // BT
NG = NB + NE - 1
CH = 256
NCH = NT // CH
NWK = 32
TPW = NT // NWK


def _gate_body(x_ref, wg_ref, w_ref, p_ref, cnt_ref):
    x = x_ref[...]
    gate = lax.dot_general(x, wg_ref[...], (((1,), (0,)), ((), ())),
                           preferred_element_type=jnp.float32)
    iota = lax.broadcasted_iota(jnp.int32, gate.shape, 1)
    v1 = jnp.max(gate, axis=1, keepdims=True)
    i1 = jnp.min(jnp.where(gate >= v1, iota, NE), axis=1, keepdims=True)
    g2 = jnp.where(iota == i1, -jnp.inf, gate)
    v2 = jnp.max(g2, axis=1, keepdims=True)
    i2 = jnp.min(jnp.where(g2 >= v2, iota, NE), axis=1, keepdims=True)
    p1v = 1.0 / (1.0 + jnp.exp(v2 - v1))
    estar = jnp.maximum(i1, i2)
    wstar = jnp.where(i1 >= i2, p1v, 1.0 - p1v)
    onehot = (iota == estar).astype(jnp.float32)

    tri = (lax.broadcasted_iota(jnp.int32, (CH, CH), 0)
           >= lax.broadcasted_iota(jnp.int32, (CH, CH), 1)).astype(jnp.float32)
    prefs = []
    run_tot = jnp.zeros((1, NE), jnp.float32)
    for k in range(NCH):
        blk = onehot[k * CH:(k + 1) * CH, :]
        incl = lax.dot_general(tri, blk, (((1,), (0,)), ((), ())),
                               preferred_element_type=jnp.float32)
        prefs.append(incl + run_tot)
        run_tot = run_tot + incl[CH - 1:CH, :]
    pref = jnp.concatenate(prefs, axis=0)

    stri = (lax.broadcasted_iota(jnp.int32, (NE, NE), 0)
            < lax.broadcasted_iota(jnp.int32, (NE, NE), 1)).astype(jnp.float32)
    starts = lax.dot_general(run_tot, stri, (((1,), (0,)), ((), ())),
                             precision=lax.Precision.HIGHEST,
                             preferred_element_type=jnp.float32)
    pos = jnp.sum(onehot * (pref + starts), axis=1, keepdims=True) - 1.0

    w_ref[...] = wstar
    p_ref[...] = (pos + 0.5).astype(jnp.int32)
    cnt_ref[...] = (run_tot + 0.5).astype(jnp.int32)


def _gating(x, Wg):
    return pl.pallas_call(
        _gate_body,
        in_specs=[
            pl.BlockSpec((NT, DM), lambda: (0, 0)),
            pl.BlockSpec((DM, NE), lambda: (0, 0)),
        ],
        out_specs=[
            pl.BlockSpec((NT, 1), lambda: (0, 0)),
            pl.BlockSpec((NT, 1), lambda: (0, 0)),
            pl.BlockSpec((1, NE), lambda: (0, 0)),
        ],
        out_shape=[
            jax.ShapeDtypeStruct((NT, 1), jnp.float32),
            jax.ShapeDtypeStruct((NT, 1), jnp.int32),
            jax.ShapeDtypeStruct((1, NE), jnp.int32),
        ],
    )(x, Wg)


def _work_list(counts):
    c = counts.reshape(NE)
    ends = jnp.cumsum(c)
    starts = ends - c
    bs = starts // BT
    be = jnp.where(c > 0, (ends + BT - 1) // BT, bs)
    n = be - bs
    icum = jnp.cumsum(n)
    ibefore = icum - n
    g = jnp.arange(NG)
    item_e = jnp.searchsorted(icum, g, side="right")
    e_c = jnp.minimum(item_e, NE - 1)
    item_b = jnp.where(item_e < NE, bs[e_c] + (g - ibefore[e_c]), NB - 1)
    lo = jnp.clip(starts[e_c] - item_b * BT, 0, BT)
    hi = jnp.clip(ends[e_c] - item_b * BT, 0, BT)
    hi = jnp.where(item_e < NE, hi, 0)
    return jnp.stack([item_b, e_c, lo, hi]).astype(jnp.int32)


def _sc_scatter(x, ws, pos):
    mesh = plsc.VectorSubcoreMesh(core_axis_name="c", subcore_axis_name="s")

    @functools.partial(
        pl.kernel, mesh=mesh,
        out_type=[
            jax.ShapeDtypeStruct((NT, DM), jnp.float32),
            jax.ShapeDtypeStruct((NT,), jnp.float32),
        ],
        scratch_types=[
            pltpu.VMEM((TPW,), jnp.int32),
            pltpu.VMEM((TPW, DM), jnp.float32),
            pltpu.VMEM((TPW,), jnp.float32),
            pltpu.SemaphoreType.DMA,
        ],
    )
    def body(x_hbm, ws_hbm, pos_hbm, xs_hbm, wss_hbm, idx_v, rows_v, w_v, sem):
        wid = lax.axis_index("s") * 2 + lax.axis_index("c")
        base = wid * TPW
        pltpu.sync_copy(pos_hbm.at[pl.ds(base, TPW)], idx_v)
        pltpu.sync_copy(x_hbm.at[pl.ds(base, TPW)], rows_v)
        pltpu.sync_copy(ws_hbm.at[pl.ds(base, TPW)], w_v)
        pltpu.async_copy(rows_v, xs_hbm.at[idx_v], sem).wait()
        pltpu.async_copy(w_v, wss_hbm.at[idx_v], sem).wait()

    return body(x, ws, pos)


def _group_body(meta_ref, xs_ref, w_ref, b_ref, ws_ref, o_ref):
    g = pl.program_id(0)
    blk = meta_ref[0, g]
    lo = meta_ref[2, g]
    hi = meta_ref[3, g]
    prev = meta_ref[0, jnp.maximum(g - 1, 0)]
    first = jnp.logical_or(g == 0, blk != prev)
    acc = lax.dot_general(xs_ref[...], w_ref[0], (((1,), (1,)), ((), ())),
                          preferred_element_type=jnp.float32)
    contrib = ws_ref[...] * (acc + b_ref[0])
    riota = lax.broadcasted_iota(jnp.int32, (BT, 1), 0)
    mask = jnp.logical_and(riota >= lo, riota < hi)
    contrib = jnp.where(mask, contrib, 0.0)

    @pl.when(first)
    def _():
        o_ref[...] = contrib

    @pl.when(jnp.logical_not(first))
    def _():
        o_ref[...] = o_ref[...] + contrib


def _grouped_matmul(meta, xs, W, b, wss):
    grid_spec = pltpu.PrefetchScalarGridSpec(
        num_scalar_prefetch=1,
        grid=(NG,),
        in_specs=[
            pl.BlockSpec((BT, DM), lambda g, m: (m[0, g], 0)),
            pl.BlockSpec((1, DM, DM), lambda g, m: (m[1, g], 0, 0)),
            pl.BlockSpec((1, 1, DM), lambda g, m: (m[1, g], 0, 0)),
            pl.BlockSpec((BT, 1), lambda g, m: (m[0, g], 0)),
        ],
        out_specs=pl.BlockSpec((BT, DM), lambda g, m: (m[0, g], 0)),
    )
    return pl.pallas_call(
        _group_body,
        grid_spec=grid_spec,
        out_shape=jax.ShapeDtypeStruct((NT, DM), jnp.float32),
    )(meta, xs, W, b, wss)


def _sc_gather(ys, pos):
    mesh = plsc.VectorSubcoreMesh(core_axis_name="c", subcore_axis_name="s")

    @functools.partial(
        pl.kernel, mesh=mesh,
        out_type=jax.ShapeDtypeStruct((NT, DM), jnp.float32),
        scratch_types=[
            pltpu.VMEM((TPW,), jnp.int32),
            pltpu.VMEM((TPW, DM), jnp.float32),
            pltpu.SemaphoreType.DMA,
        ],
    )
    def body(ys_hbm, pos_hbm, out_hbm, idx_v, rows_v, sem):
        wid = lax.axis_index("s") * 2 + lax.axis_index("c")
        base = wid * TPW
        pltpu.sync_copy(pos_hbm.at[pl.ds(base, TPW)], idx_v)
        pltpu.async_copy(ys_hbm.at[idx_v], rows_v, sem).wait()
        pltpu.sync_copy(rows_v, out_hbm.at[pl.ds(base, TPW)])

    return body(ys, pos)


def kernel(x, Wg, W, b):
    ws, pos, counts = _gating(x, Wg)
    meta = _work_list(counts)
    pos1 = pos.reshape(NT)
    xs, wss = _sc_scatter(x, ws.reshape(NT), pos1)
    ys = _grouped_matmul(meta, xs, W, b.reshape(NE, 1, DM), wss.reshape(NT, 1))
    return _sc_gather(ys, pos1)


# speedup vs baseline: 2.0764x; 2.0764x over previous
import jax
import jax.numpy as jnp
from jax import lax
from jax.experimental import pallas as pl
from jax.experimental.pallas import tpu as pltpu

DM = 768
NE = 8
NT = 2048


def _moe_body(x_ref, wg_ref, w_ref, b_ref, o_ref, xbf_ref, est_ref, wst_ref):
    e = pl.program_id(0)

    @pl.when(e == 0)
    def _():
        x = x_ref[...]
        gate = lax.dot_general(x, wg_ref[...], (((1,), (0,)), ((), ())),
                               preferred_element_type=jnp.float32)
        iota = lax.broadcasted_iota(jnp.int32, gate.shape, 1)
        v1 = jnp.max(gate, axis=1, keepdims=True)
        i1 = jnp.min(jnp.where(gate >= v1, iota, NE), axis=1, keepdims=True)
        g2 = jnp.where(iota == i1, -jnp.inf, gate)
        v2 = jnp.max(g2, axis=1, keepdims=True)
        i2 = jnp.min(jnp.where(g2 >= v2, iota, NE), axis=1, keepdims=True)
        p1v = 1.0 / (1.0 + jnp.exp(v2 - v1))
        est_ref[...] = jnp.maximum(i1, i2)
        wst_ref[...] = jnp.where(i1 >= i2, p1v, 1.0 - p1v)
        xbf_ref[...] = x.astype(jnp.bfloat16)

    eo = lax.dot_general(xbf_ref[...], w_ref[0].astype(jnp.bfloat16),
                         (((1,), (1,)), ((), ())),
                         preferred_element_type=jnp.float32) + b_ref[0]
    contrib = wst_ref[...] * eo
    sel = est_ref[...] == e

    @pl.when(e == 0)
    def _():
        o_ref[...] = jnp.where(sel, contrib, 0.0)

    @pl.when(e > 0)
    def _():
        o_ref[...] = jnp.where(sel, contrib, o_ref[...])


def kernel(x, Wg, W, b):
    return pl.pallas_call(
        _moe_body,
        grid=(NE,),
        in_specs=[
            pl.BlockSpec((NT, DM), lambda e: (0, 0)),
            pl.BlockSpec((DM, NE), lambda e: (0, 0)),
            pl.BlockSpec((1, DM, DM), lambda e: (e, 0, 0)),
            pl.BlockSpec((1, 1, DM), lambda e: (e, 0, 0)),
        ],
        out_specs=pl.BlockSpec((NT, DM), lambda e: (0, 0)),
        out_shape=jax.ShapeDtypeStruct((NT, DM), jnp.float32),
        scratch_shapes=[
            pltpu.VMEM((NT, DM), jnp.bfloat16),
            pltpu.VMEM((NT, 1), jnp.int32),
            pltpu.VMEM((NT, 1), jnp.float32),
        ],
    )(x, Wg, W, b.reshape(NE, 1, DM))


# speedup vs baseline: 2.5058x; 1.2068x over previous
import jax
import jax.numpy as jnp
from jax import lax
from jax.experimental import pallas as pl
from jax.experimental.pallas import tpu as pltpu

DM = 768
NE = 8
NT = 2048
BT = 256


def _moe_body(x_ref, wg_ref, w_ref, b_ref, o_ref, wbf_ref):
    i = pl.program_id(0)

    @pl.when(i == 0)
    def _():
        wbf_ref[...] = w_ref[...].astype(jnp.bfloat16)

    x = x_ref[...]
    gate = lax.dot_general(x, wg_ref[...], (((1,), (0,)), ((), ())),
                           preferred_element_type=jnp.float32)
    iota = lax.broadcasted_iota(jnp.int32, gate.shape, 1)
    v1 = jnp.max(gate, axis=1, keepdims=True)
    i1 = jnp.min(jnp.where(gate >= v1, iota, NE), axis=1, keepdims=True)
    g2 = jnp.where(iota == i1, -jnp.inf, gate)
    v2 = jnp.max(g2, axis=1, keepdims=True)
    i2 = jnp.min(jnp.where(g2 >= v2, iota, NE), axis=1, keepdims=True)
    p1 = 1.0 / (1.0 + jnp.exp(v2 - v1))
    estar = jnp.maximum(i1, i2)
    wstar = jnp.where(i1 >= i2, p1, 1.0 - p1)
    xbf = x.astype(jnp.bfloat16)
    out = jnp.zeros_like(x)
    for e in range(NE):
        eo = lax.dot_general(
            xbf, wbf_ref[e], (((1,), (1,)), ((), ())),
            preferred_element_type=jnp.float32) + b_ref[e][None, :]
        out = jnp.where(estar == e, wstar * eo, out)
    o_ref[...] = out


def kernel(x, Wg, W, b):
    return pl.pallas_call(
        _moe_body,
        grid=(NT // BT,),
        in_specs=[
            pl.BlockSpec((BT, DM), lambda i: (i, 0)),
            pl.BlockSpec((DM, NE), lambda i: (0, 0)),
            pl.BlockSpec((NE, DM, DM), lambda i: (0, 0, 0)),
            pl.BlockSpec((NE, DM), lambda i: (0, 0)),
        ],
        out_specs=pl.BlockSpec((BT, DM), lambda i: (i, 0)),
        out_shape=jax.ShapeDtypeStruct((NT, DM), jnp.float32),
        scratch_shapes=[pltpu.VMEM((NE, DM, DM), jnp.bfloat16)],
    )(x, Wg, W, b)
